# baseline (speedup 1.0000x reference)
import functools

import jax
import jax.numpy as jnp
from jax import lax
from jax.experimental import pallas as pl
from jax.experimental.pallas import tpu as pltpu
from jax.experimental.pallas import tpu_sc as plsc

N = 10000
E = 320000
D = 128

NC = 2
NS = 16
NW = NC * NS
LANES = 16

CHUNK = 128
E_PAD = 327680
E_PER_W = E_PAD // NW
STEPS = E_PER_W // CHUNK
NPAIR = STEPS // 2
NP = 10240
ROWS_PER_TILE = NP // NS


def _sc_aggregate(feat_hbm, src_hbm, dst_hbm, partial_hbm, deg_hbm,
                  srcb, dstb, rows, hist_v, acc_sh, semg, semi):
    c = lax.axis_index("c")
    s = lax.axis_index("s")
    wid = c * NS + s
    ebase = wid * E_PER_W

    zeros16 = jnp.zeros((LANES,), jnp.float32)
    ones16 = jnp.ones((LANES,), jnp.float32)

    def zero_rows(k, _):
        i = k // (D // LANES)
        j = k % (D // LANES)
        rows[0][i, pl.ds(j * LANES, LANES)] = zeros16
        return 0
    lax.fori_loop(0, CHUNK * (D // LANES), zero_rows, 0)

    def zero_hist(k, _):
        hist_v[pl.ds(k * LANES, LANES)] = zeros16
        return 0
    lax.fori_loop(0, NP // LANES, zero_hist, 0)

    for t in range(ROWS_PER_TILE // CHUNK):
        pltpu.sync_copy(rows[0], acc_sh.at[pl.ds(s * ROWS_PER_TILE + t * CHUNK, CHUNK)])

    plsc.subcore_barrier()

    def hist_update(p):
        for j in range(CHUNK // LANES):
            idx = dstb[p][pl.ds(j * LANES, LANES)]
            plsc.addupdate_scatter(hist_v, [idx], ones16)

    def prefetch_idx(i, p):
        pltpu.async_copy(src_hbm.at[pl.ds(ebase + i * CHUNK, CHUNK)], srcb[p], semi[p])
        pltpu.async_copy(dst_hbm.at[pl.ds(ebase + i * CHUNK, CHUNK)], dstb[p], semi[p])

    def wait_idx(p):
        pltpu.make_async_copy(src_hbm.at[pl.ds(0, CHUNK)], srcb[p], semi[p]).wait()
        pltpu.make_async_copy(dst_hbm.at[pl.ds(0, CHUNK)], dstb[p], semi[p]).wait()

    def step(i, p, issue_next, prefetch_next):
        pn = p ^ 1
        if issue_next:
            wait_idx(pn)
            pltpu.async_copy(feat_hbm.at[srcb[pn]], rows[pn], semg[pn])
        pltpu.make_async_copy(feat_hbm.at[srcb[p]], rows[p], semg[p]).wait()
        pltpu.sync_copy(rows[p], acc_sh.at[dstb[p]], add=True)
        hist_update(p)
        if prefetch_next:
            prefetch_idx(i + 2, p)

    def simple_step(i, _):
        pltpu.sync_copy(src_hbm.at[pl.ds(ebase + i * CHUNK, CHUNK)], srcb[0])
        pltpu.sync_copy(dst_hbm.at[pl.ds(ebase + i * CHUNK, CHUNK)], dstb[0])
        pltpu.async_copy(feat_hbm.at[srcb[0]], rows[0], semg[0]).wait()
        pltpu.sync_copy(rows[0], acc_sh.at[dstb[0]], add=True)
        hist_update(0)
        return 0

    lax.fori_loop(0, STEPS, simple_step, 0)

    plsc.subcore_barrier()

    for t in range(ROWS_PER_TILE // CHUNK):
        r0 = s * ROWS_PER_TILE + t * CHUNK
        pltpu.sync_copy(acc_sh.at[pl.ds(r0, CHUNK)], partial_hbm.at[c, pl.ds(r0, CHUNK)])
    pltpu.sync_copy(hist_v.at[pl.ds(0, N)], deg_hbm.at[pl.ds(wid * N, N)])


_sc_call = functools.partial(
    pl.kernel,
    out_type=[
        jax.ShapeDtypeStruct((NC, NP, D), jnp.float32),
        jax.ShapeDtypeStruct((NW * N,), jnp.float32),
    ],
    mesh=plsc.VectorSubcoreMesh(core_axis_name="c", subcore_axis_name="s"),
    compiler_params=pltpu.CompilerParams(needs_layout_passes=False),
    scratch_types=[
        [pltpu.VMEM((CHUNK,), jnp.int32)] * 2,
        [pltpu.VMEM((CHUNK,), jnp.int32)] * 2,
        [pltpu.VMEM((CHUNK, D), jnp.float32)] * 2,
        pltpu.VMEM((NP,), jnp.float32),
        pltpu.VMEM_SHARED((NP, D), jnp.float32),
        [pltpu.SemaphoreType.DMA] * 2,
        [pltpu.SemaphoreType.DMA] * 2,
    ],
)(_sc_aggregate)


ROWS_BLK = 400


def _tc_combine(partial_ref, deg_ref, w_ref, out_ref):
    p = partial_ref[...]
    summed = p[0] + p[1]
    deg = jnp.sum(deg_ref[...], axis=1)
    deg = jnp.maximum(deg, 1.0)
    mean = summed / deg[:, None]
    out_ref[...] = lax.dot_general(
        mean, w_ref[...], (((1,), (1,)), ((), ())),
        preferred_element_type=jnp.float32)


def _combine(partial, deg, W):
    return pl.pallas_call(
        _tc_combine,
        grid=(N // ROWS_BLK,),
        in_specs=[
            pl.BlockSpec((NC, ROWS_BLK, D), lambda i: (0, i, 0)),
            pl.BlockSpec((ROWS_BLK, NW), lambda i: (i, 0)),
            pl.BlockSpec((D, D), lambda i: (0, 0)),
        ],
        out_specs=pl.BlockSpec((ROWS_BLK, D), lambda i: (i, 0)),
        out_shape=jax.ShapeDtypeStruct((N, D), jnp.float32),
    )(partial, deg, W)


def kernel(feat, edge_index, W):
    src = edge_index[0]
    dst = edge_index[1]
    pad = E_PAD - E
    src = jnp.concatenate([src, jnp.zeros((pad,), jnp.int32)])
    dst = jnp.concatenate([dst, jnp.full((pad,), N, jnp.int32)])
    partial, deg = _sc_call(feat, src, dst)
    return _combine(partial, deg.reshape(NW, N).T, W)


# speedup vs baseline: 3.4115x; 3.4115x over previous
import functools

import jax
import jax.numpy as jnp
from jax import lax
from jax.experimental import pallas as pl
from jax.experimental.pallas import tpu as pltpu
from jax.experimental.pallas import tpu_sc as plsc

N = 10000
E = 320000
D = 128

NC = 2
NS = 16
NW = NC * NS
LANES = 16

CHUNK = 80
E_PER_W = E // NW
STEPS = E_PER_W // CHUNK
NP = 10240
ROWS_PER_TILE = NP // NS
ZROWS = 128


def _sc_aggregate(feat_hbm, src_hbm, dst_hbm, partial_hbm, deg_hbm,
                  srcb, dstb, rows, hist_v, zbuf_v, acc_sh, semg, semi):
    c = lax.axis_index("c")
    s = lax.axis_index("s")
    wid = c * NS + s
    ebase = wid * E_PER_W

    zeros16 = jnp.zeros((LANES,), jnp.float32)
    ones16 = jnp.ones((LANES,), jnp.float32)

    def zero_zbuf(k, _):
        i = k // (D // LANES)
        j = k % (D // LANES)
        zbuf_v[i, pl.ds(j * LANES, LANES)] = zeros16
        return 0
    lax.fori_loop(0, ZROWS * (D // LANES), zero_zbuf, 0)

    def zero_hist(k, _):
        hist_v[pl.ds(k * LANES, LANES)] = zeros16
        return 0
    lax.fori_loop(0, N // LANES, zero_hist, 0)

    for t in range(ROWS_PER_TILE // ZROWS):
        pltpu.sync_copy(zbuf_v, acc_sh.at[pl.ds(s * ROWS_PER_TILE + t * ZROWS, ZROWS)])

    plsc.subcore_barrier()

    def hist_update(p):
        for j in range(CHUNK // LANES):
            idx = dstb[p][pl.ds(j * LANES, LANES)]
            plsc.addupdate_scatter(hist_v, [idx], ones16)

    def prefetch_idx(i, p):
        pltpu.async_copy(src_hbm.at[pl.ds(ebase + i * CHUNK, CHUNK)], srcb[p], semi[p])
        pltpu.async_copy(dst_hbm.at[pl.ds(ebase + i * CHUNK, CHUNK)], dstb[p], semi[p])

    def wait_idx(p):
        pltpu.make_async_copy(src_hbm.at[pl.ds(0, CHUNK)], srcb[p], semi[p]).wait()
        pltpu.make_async_copy(dst_hbm.at[pl.ds(0, CHUNK)], dstb[p], semi[p]).wait()

    def step(i, p, issue_next, prefetch_next):
        pn = p ^ 1
        if issue_next:
            wait_idx(pn)
            pltpu.async_copy(feat_hbm.at[srcb[pn]], rows[pn], semg[pn])
        pltpu.make_async_copy(feat_hbm.at[srcb[p]], rows[p], semg[p]).wait()
        pltpu.sync_copy(rows[p], acc_sh.at[dstb[p]], add=True)
        hist_update(p)
        if prefetch_next:
            prefetch_idx(i + 2, p)

    pltpu.sync_copy(src_hbm.at[pl.ds(ebase, CHUNK)], srcb[0])
    pltpu.sync_copy(dst_hbm.at[pl.ds(ebase, CHUNK)], dstb[0])
    pltpu.async_copy(feat_hbm.at[srcb[0]], rows[0], semg[0])
    prefetch_idx(1, 1)

    def pair(k, _):
        i0 = 2 * k
        step(i0, 0, True, True)
        step(i0 + 1, 1, True, True)
        return 0

    lax.fori_loop(0, (STEPS - 3) // 2, pair, 0)

    step(STEPS - 3, 0, True, True)
    step(STEPS - 2, 1, True, False)
    step(STEPS - 1, 0, False, False)

    plsc.subcore_barrier()

    for t in range(ROWS_PER_TILE // ZROWS):
        r0 = s * ROWS_PER_TILE + t * ZROWS
        pltpu.sync_copy(acc_sh.at[pl.ds(r0, ZROWS)], partial_hbm.at[c, pl.ds(r0, ZROWS)])
    pltpu.sync_copy(hist_v, deg_hbm.at[pl.ds(wid * N, N)])


_sc_call = functools.partial(
    pl.kernel,
    out_type=[
        jax.ShapeDtypeStruct((NC, NP, D), jnp.float32),
        jax.ShapeDtypeStruct((NW * N,), jnp.float32),
    ],
    mesh=plsc.VectorSubcoreMesh(core_axis_name="c", subcore_axis_name="s"),
    compiler_params=pltpu.CompilerParams(needs_layout_passes=False),
    scratch_types=[
        [pltpu.VMEM((CHUNK,), jnp.int32)] * 2,
        [pltpu.VMEM((CHUNK,), jnp.int32)] * 2,
        [pltpu.VMEM((CHUNK, D), jnp.float32)] * 2,
        pltpu.VMEM((N,), jnp.float32),
        pltpu.VMEM((ZROWS, D), jnp.float32),
        pltpu.VMEM_SHARED((NP, D), jnp.float32),
        [pltpu.SemaphoreType.DMA] * 2,
        [pltpu.SemaphoreType.DMA] * 2,
    ],
)(_sc_aggregate)


ROWS_BLK = 400


def _tc_combine(partial_ref, deg_ref, w_ref, out_ref):
    p = partial_ref[...]
    summed = p[0] + p[1]
    deg = jnp.sum(deg_ref[...], axis=1)
    deg = jnp.maximum(deg, 1.0)
    mean = summed / deg[:, None]
    out_ref[...] = lax.dot_general(
        mean, w_ref[...], (((1,), (1,)), ((), ())),
        preferred_element_type=jnp.float32)


def _combine(partial, deg, W):
    return pl.pallas_call(
        _tc_combine,
        grid=(N // ROWS_BLK,),
        in_specs=[
            pl.BlockSpec((NC, ROWS_BLK, D), lambda i: (0, i, 0)),
            pl.BlockSpec((ROWS_BLK, NW), lambda i: (i, 0)),
            pl.BlockSpec((D, D), lambda i: (0, 0)),
        ],
        out_specs=pl.BlockSpec((ROWS_BLK, D), lambda i: (i, 0)),
        out_shape=jax.ShapeDtypeStruct((N, D), jnp.float32),
    )(partial, deg, W)


def kernel(feat, edge_index, W):
    src = edge_index[0]
    dst = edge_index[1]
    partial, deg = _sc_call(feat, src, dst)
    return _combine(partial, deg.reshape(NW, N).T, W)


# speedup vs baseline: 3.5791x; 1.0491x over previous
import functools

import jax
import jax.numpy as jnp
from jax import lax
from jax.experimental import pallas as pl
from jax.experimental.pallas import tpu as pltpu
from jax.experimental.pallas import tpu_sc as plsc

N = 10000
E = 320000
D = 128

NC = 2
NS = 16
NW = NC * NS
LANES = 16

CHUNK = 80
E_PER_W = E // NW
STEPS = E_PER_W // CHUNK
DG = 3
NP = 10240
ROWS_PER_TILE = NP // NS


def _sc_aggregate(feat_hbm, src_hbm, dst_hbm, partial_hbm, deg_hbm,
                  srcb, dstb, rows, hist_v, acc_sh, semg, semi):
    c = lax.axis_index("c")
    s = lax.axis_index("s")
    wid = c * NS + s
    ebase = wid * E_PER_W

    zeros16 = jnp.zeros((LANES,), jnp.float32)
    ones16 = jnp.ones((LANES,), jnp.float32)

    def zero_rows0(k, _):
        i = k // (D // LANES)
        j = k % (D // LANES)
        rows[0][i, pl.ds(j * LANES, LANES)] = zeros16
        return 0
    lax.fori_loop(0, CHUNK * (D // LANES), zero_rows0, 0)

    def zero_hist(k, _):
        hist_v[pl.ds(k * LANES, LANES)] = zeros16
        return 0
    lax.fori_loop(0, N // LANES, zero_hist, 0)

    for t in range(ROWS_PER_TILE // CHUNK):
        pltpu.sync_copy(rows[0], acc_sh.at[pl.ds(s * ROWS_PER_TILE + t * CHUNK, CHUNK)])

    plsc.subcore_barrier()

    def prefetch_idx(i, p):
        pltpu.async_copy(src_hbm.at[pl.ds(ebase + i * CHUNK, CHUNK)], srcb[p], semi[p])
        pltpu.async_copy(dst_hbm.at[pl.ds(ebase + i * CHUNK, CHUNK)], dstb[p], semi[p])

    def wait_idx(p):
        pltpu.make_async_copy(src_hbm.at[pl.ds(0, CHUNK)], srcb[p], semi[p]).wait()
        pltpu.make_async_copy(dst_hbm.at[pl.ds(0, CHUNK)], dstb[p], semi[p]).wait()

    def step(i, p, issue_ahead, prefetch_ahead):
        pa = (p + DG - 1) % DG
        if issue_ahead:
            wait_idx(pa)
            pltpu.async_copy(feat_hbm.at[srcb[pa]], rows[pa], semg[pa])
        pltpu.make_async_copy(feat_hbm.at[srcb[p]], rows[p], semg[p]).wait()
        pltpu.sync_copy(rows[p], acc_sh.at[dstb[p]], add=True)
        for j in range(CHUNK // LANES):
            idx = dstb[p][pl.ds(j * LANES, LANES)]
            plsc.addupdate_scatter(hist_v, [idx], ones16)
        if prefetch_ahead:
            prefetch_idx(i + DG, p)

    for j in range(DG - 1):
        pltpu.sync_copy(src_hbm.at[pl.ds(ebase + j * CHUNK, CHUNK)], srcb[j])
        pltpu.sync_copy(dst_hbm.at[pl.ds(ebase + j * CHUNK, CHUNK)], dstb[j])
        pltpu.async_copy(feat_hbm.at[srcb[j]], rows[j], semg[j])
    prefetch_idx(DG - 1, DG - 1)

    NGRP = (STEPS - DG) // DG

    def group(g, _):
        i0 = g * DG
        for k in range(DG):
            step(i0 + k, k, True, True)
        return 0

    lax.fori_loop(0, NGRP, group, 0)

    for i in range(NGRP * DG, STEPS):
        step(i, i % DG, i + DG - 1 <= STEPS - 1, i + DG <= STEPS - 1)

    plsc.subcore_barrier()

    for t in range(ROWS_PER_TILE // CHUNK):
        r0 = s * ROWS_PER_TILE + t * CHUNK
        pltpu.sync_copy(acc_sh.at[pl.ds(r0, CHUNK)], partial_hbm.at[c, pl.ds(r0, CHUNK)])
    pltpu.sync_copy(hist_v, deg_hbm.at[pl.ds(wid * N, N)])


_sc_call = functools.partial(
    pl.kernel,
    out_type=[
        jax.ShapeDtypeStruct((NC, NP, D), jnp.float32),
        jax.ShapeDtypeStruct((NW * N,), jnp.float32),
    ],
    mesh=plsc.VectorSubcoreMesh(core_axis_name="c", subcore_axis_name="s"),
    compiler_params=pltpu.CompilerParams(needs_layout_passes=False),
    scratch_types=[
        [pltpu.VMEM((CHUNK,), jnp.int32)] * DG,
        [pltpu.VMEM((CHUNK,), jnp.int32)] * DG,
        [pltpu.VMEM((CHUNK, D), jnp.float32)] * DG,
        pltpu.VMEM((N,), jnp.float32),
        pltpu.VMEM_SHARED((NP, D), jnp.float32),
        [pltpu.SemaphoreType.DMA] * DG,
        [pltpu.SemaphoreType.DMA] * DG,
    ],
)(_sc_aggregate)


ROWS_BLK = 400


def _tc_combine(partial_ref, deg_ref, w_ref, out_ref):
    p = partial_ref[...]
    summed = p[0] + p[1]
    deg = jnp.sum(deg_ref[...], axis=1)
    deg = jnp.maximum(deg, 1.0)
    mean = summed / deg[:, None]
    out_ref[...] = lax.dot_general(
        mean, w_ref[...], (((1,), (1,)), ((), ())),
        preferred_element_type=jnp.float32)


def _combine(partial, deg, W):
    return pl.pallas_call(
        _tc_combine,
        grid=(N // ROWS_BLK,),
        in_specs=[
            pl.BlockSpec((NC, ROWS_BLK, D), lambda i: (0, i, 0)),
            pl.BlockSpec((ROWS_BLK, NW), lambda i: (i, 0)),
            pl.BlockSpec((D, D), lambda i: (0, 0)),
        ],
        out_specs=pl.BlockSpec((ROWS_BLK, D), lambda i: (i, 0)),
        out_shape=jax.ShapeDtypeStruct((N, D), jnp.float32),
    )(partial, deg, W)


def kernel(feat, edge_index, W):
    src = edge_index[0]
    dst = edge_index[1]
    partial, deg = _sc_call(feat, src, dst)
    return _combine(partial, deg.reshape(NW, N).T, W)


# speedup vs baseline: 3.7302x; 1.0422x over previous
import functools

import jax
import jax.numpy as jnp
from jax import lax
from jax.experimental import pallas as pl
from jax.experimental.pallas import tpu as pltpu
from jax.experimental.pallas import tpu_sc as plsc

N = 10000
E = 320000
D = 128

NC = 2
NS = 16
NW = NC * NS
LANES = 16

CHUNK = 80
E_PER_W = E // NW
STEPS = E_PER_W // CHUNK
LEAD = 1
DR = 3
DI = 4
UNROLL = 12
MID0 = DR - LEAD
NMID = ((STEPS - LEAD - 2) - MID0 + 1) // UNROLL * UNROLL
NP = 10240
ROWS_PER_TILE = NP // NS
WB = 40


def _sc_aggregate(feat_hbm, src_hbm, dst_hbm, partial_hbm, deg_hbm,
                  srcb, dstb, rows, hist_v, acc_sh, semg, semi, semsc):
    c = lax.axis_index("c")
    s = lax.axis_index("s")
    wid = c * NS + s
    ebase = wid * E_PER_W

    zeros16 = jnp.zeros((LANES,), jnp.float32)
    ones16 = jnp.ones((LANES,), jnp.float32)

    def zero_rows0(k, _):
        i = k // (D // LANES)
        j = k % (D // LANES)
        rows[0][i, pl.ds(j * LANES, LANES)] = zeros16
        return 0
    lax.fori_loop(0, CHUNK * (D // LANES), zero_rows0, 0)

    def zero_hist(k, _):
        hist_v[pl.ds(k * LANES, LANES)] = zeros16
        return 0
    lax.fori_loop(0, N // LANES, zero_hist, 0)

    for t in range(ROWS_PER_TILE // WB):
        pltpu.sync_copy(rows[0].at[pl.ds(0, WB)],
                        acc_sh.at[pl.ds(s * ROWS_PER_TILE + t * WB, WB)])

    plsc.subcore_barrier()

    def prefetch_idx(i, q):
        pltpu.async_copy(src_hbm.at[pl.ds(ebase + i * CHUNK, CHUNK)], srcb[q], semi[q])
        pltpu.async_copy(dst_hbm.at[pl.ds(ebase + i * CHUNK, CHUNK)], dstb[q], semi[q])

    def wait_idx(q):
        pltpu.make_async_copy(src_hbm.at[pl.ds(0, CHUNK)], srcb[q], semi[q]).wait()
        pltpu.make_async_copy(dst_hbm.at[pl.ds(0, CHUNK)], dstb[q], semi[q]).wait()

    def step(i, si, wait_scatter, issue_ahead, prefetch_ahead):
        p = si % DR
        q = si % DI
        gs = (si + LEAD) % DR
        gq = (si + LEAD) % DI
        if issue_ahead:
            if wait_scatter:
                pltpu.make_async_copy(rows[gs], acc_sh.at[pl.ds(0, CHUNK)], semsc[gs]).wait()
            wait_idx(gq)
            pltpu.async_copy(feat_hbm.at[srcb[gq]], rows[gs], semg[gs])
        pltpu.make_async_copy(feat_hbm.at[srcb[q]], rows[p], semg[p]).wait()
        pltpu.async_copy(rows[p], acc_sh.at[dstb[q]], semsc[p], add=True)
        for j in range(CHUNK // LANES):
            idx = dstb[q][pl.ds(j * LANES, LANES)]
            plsc.addupdate_scatter(hist_v, [idx], ones16)
        if prefetch_ahead:
            prefetch_idx(i + LEAD + 1, (si + LEAD + 1) % DI)

    for j in range(LEAD):
        pltpu.sync_copy(src_hbm.at[pl.ds(ebase + j * CHUNK, CHUNK)], srcb[j])
        pltpu.sync_copy(dst_hbm.at[pl.ds(ebase + j * CHUNK, CHUNK)], dstb[j])
        pltpu.async_copy(feat_hbm.at[srcb[j]], rows[j], semg[j])
    prefetch_idx(LEAD, LEAD)

    for i in range(MID0):
        step(i, i, False, True, True)

    def group(gidx, _):
        i0 = MID0 + gidx * UNROLL
        for k in range(UNROLL):
            step(i0 + k, MID0 + k, True, True, True)
        return 0

    lax.fori_loop(0, NMID // UNROLL, group, 0)

    for i in range(MID0 + NMID, STEPS):
        step(i, i, i >= DR - LEAD, i + LEAD <= STEPS - 1, i + LEAD + 1 <= STEPS - 1)

    for i in range(STEPS - DR, STEPS):
        p = i % DR
        pltpu.make_async_copy(rows[p], acc_sh.at[pl.ds(0, CHUNK)], semsc[p]).wait()

    plsc.subcore_barrier()

    for t in range(ROWS_PER_TILE // WB):
        r0 = s * ROWS_PER_TILE + t * WB
        pltpu.sync_copy(acc_sh.at[pl.ds(r0, WB)], partial_hbm.at[c, pl.ds(r0, WB)])
    pltpu.sync_copy(hist_v, deg_hbm.at[pl.ds(wid * N, N)])


_sc_call = functools.partial(
    pl.kernel,
    out_type=[
        jax.ShapeDtypeStruct((NC, NP, D), jnp.float32),
        jax.ShapeDtypeStruct((NW * N,), jnp.float32),
    ],
    mesh=plsc.VectorSubcoreMesh(core_axis_name="c", subcore_axis_name="s"),
    compiler_params=pltpu.CompilerParams(needs_layout_passes=False),
    scratch_types=[
        [pltpu.VMEM((CHUNK,), jnp.int32)] * DI,
        [pltpu.VMEM((CHUNK,), jnp.int32)] * DI,
        [pltpu.VMEM((CHUNK, D), jnp.float32)] * DR,
        pltpu.VMEM((N,), jnp.float32),
        pltpu.VMEM_SHARED((NP, D), jnp.float32),
        [pltpu.SemaphoreType.DMA] * DR,
        [pltpu.SemaphoreType.DMA] * DI,
        [pltpu.SemaphoreType.DMA] * DR,
    ],
)(_sc_aggregate)


ROWS_BLK = 400


def _tc_combine(partial_ref, deg_ref, w_ref, out_ref):
    p = partial_ref[...]
    summed = p[0] + p[1]
    deg = jnp.sum(deg_ref[...], axis=1)
    deg = jnp.maximum(deg, 1.0)
    mean = summed / deg[:, None]
    out_ref[...] = lax.dot_general(
        mean, w_ref[...], (((1,), (1,)), ((), ())),
        preferred_element_type=jnp.float32)


def _combine(partial, deg, W):
    return pl.pallas_call(
        _tc_combine,
        grid=(N // ROWS_BLK,),
        in_specs=[
            pl.BlockSpec((NC, ROWS_BLK, D), lambda i: (0, i, 0)),
            pl.BlockSpec((ROWS_BLK, NW), lambda i: (i, 0)),
            pl.BlockSpec((D, D), lambda i: (0, 0)),
        ],
        out_specs=pl.BlockSpec((ROWS_BLK, D), lambda i: (i, 0)),
        out_shape=jax.ShapeDtypeStruct((N, D), jnp.float32),
    )(partial, deg, W)


def kernel(feat, edge_index, W):
    src = edge_index[0]
    dst = edge_index[1]
    partial, deg = _sc_call(feat, src, dst)
    return _combine(partial, deg.reshape(NW, N).T, W)
